# Initial kernel scaffold; baseline (speedup 1.0000x reference)
import jax
import jax.numpy as jnp
from jax.experimental import pallas as pl


def kernel(faces, face_edges, codebooks):
    raise NotImplementedError("write your pallas kernel here")


# speedup vs baseline: 1.2018x; 1.2018x over previous
import functools

import jax
import jax.numpy as jnp
from jax.experimental import pallas as pl
from jax.experimental.pallas import tpu as pltpu

_T = 2048


def _rvq_body(x_ref, cb_ref, out_ref, aux_ref):
    x = x_ref[...]
    t, d = x.shape
    num_q, k, _ = cb_ref.shape
    iota = jax.lax.broadcasted_iota(jnp.int32, (t, k), 1)
    r = x
    qout = jnp.zeros_like(x)
    aux = jnp.float32(0.0)
    for q in range(num_q):
        cb = cb_ref[q]
        cbsq = jnp.sum(cb * cb, axis=-1)
        dot = jax.lax.dot_general(
            r, cb, (((1,), (1,)), ((), ())),
            preferred_element_type=jnp.float32,
            precision=jax.lax.Precision.DEFAULT,
        )
        rsq = jnp.sum(r * r, axis=-1, keepdims=True)
        dists = rsq - 2.0 * dot + cbsq[None, :]
        m = jnp.min(dists, axis=-1, keepdims=True)
        idx = jnp.min(jnp.where(dists <= m, iota, k), axis=-1, keepdims=True)
        onehot = (iota == idx).astype(jnp.float32)
        quant = jax.lax.dot_general(
            onehot, cb, (((1,), (0,)), ((), ())),
            preferred_element_type=jnp.float32,
            precision=jax.lax.Precision.HIGHEST,
        )
        qout = qout + quant
        aux = aux + jnp.sum((quant - r) ** 2)
        r = r - quant
    out_ref[...] = qout
    aux_ref[...] = jnp.reshape(aux, (1, 1, 1))


@functools.partial(jax.jit, static_argnames=())
def kernel(faces, face_edges, codebooks):
    del face_edges
    b, n, d = faces.shape
    num_q, k, _ = codebooks.shape
    tokens = b * n
    flat = faces.reshape(tokens, d)
    grid = (tokens // _T,)
    quant, aux_partials = pl.pallas_call(
        _rvq_body,
        grid=grid,
        in_specs=[
            pl.BlockSpec((_T, d), lambda i: (i, 0)),
            pl.BlockSpec((num_q, k, d), lambda i: (0, 0, 0)),
        ],
        out_specs=[
            pl.BlockSpec((_T, d), lambda i: (i, 0)),
            pl.BlockSpec((1, 1, 1), lambda i: (i, 0, 0)),
        ],
        out_shape=[
            jax.ShapeDtypeStruct((tokens, d), jnp.float32),
            jax.ShapeDtypeStruct((grid[0], 1, 1), jnp.float32),
        ],
        compiler_params=pltpu.CompilerParams(
            dimension_semantics=("parallel",),
        ),
    )(flat, codebooks)
    aux_loss = jnp.sum(aux_partials) / jnp.float32(tokens * d)
    return quant.reshape(b, n, d), aux_loss


# speedup vs baseline: 2.4561x; 2.0436x over previous
import functools

import jax
import jax.numpy as jnp
from jax.experimental import pallas as pl
from jax.experimental.pallas import tpu as pltpu

_T = 2048


def _rvq_body(x_ref, cb_ref, out_ref, aux_ref):
    x = x_ref[...]
    t, d = x.shape
    num_q, k, _ = cb_ref.shape
    iota = jax.lax.broadcasted_iota(jnp.int32, (t, k), 1)
    r = x
    qout = jnp.zeros_like(x)
    aux = jnp.float32(0.0)
    for q in range(num_q):
        cb = cb_ref[q]
        cbsq = jnp.sum(cb * cb, axis=-1)
        dot = jax.lax.dot_general(
            r, cb, (((1,), (1,)), ((), ())),
            preferred_element_type=jnp.float32,
            precision=jax.lax.Precision.DEFAULT,
        )
        rsq = jnp.sum(r * r, axis=-1, keepdims=True)
        dists = rsq - 2.0 * dot + cbsq[None, :]
        m = jnp.min(dists, axis=-1, keepdims=True)
        idx = jnp.min(jnp.where(dists <= m, iota, k), axis=-1, keepdims=True)
        onehot = (iota == idx).astype(jnp.bfloat16)
        cb_hi = cb.astype(jnp.bfloat16)
        cb_lo = (cb - cb_hi.astype(jnp.float32)).astype(jnp.bfloat16)
        quant = jax.lax.dot_general(
            onehot, cb_hi, (((1,), (0,)), ((), ())),
            preferred_element_type=jnp.float32,
        ) + jax.lax.dot_general(
            onehot, cb_lo, (((1,), (0,)), ((), ())),
            preferred_element_type=jnp.float32,
        )
        qout = qout + quant
        aux = aux + jnp.sum((quant - r) ** 2)
        r = r - quant
    out_ref[...] = qout
    aux_ref[...] = jnp.reshape(aux, (1, 1, 1))


@functools.partial(jax.jit, static_argnames=())
def kernel(faces, face_edges, codebooks):
    del face_edges
    b, n, d = faces.shape
    num_q, k, _ = codebooks.shape
    tokens = b * n
    flat = faces.reshape(tokens, d)
    grid = (tokens // _T,)
    quant, aux_partials = pl.pallas_call(
        _rvq_body,
        grid=grid,
        in_specs=[
            pl.BlockSpec((_T, d), lambda i: (i, 0)),
            pl.BlockSpec((num_q, k, d), lambda i: (0, 0, 0)),
        ],
        out_specs=[
            pl.BlockSpec((_T, d), lambda i: (i, 0)),
            pl.BlockSpec((1, 1, 1), lambda i: (i, 0, 0)),
        ],
        out_shape=[
            jax.ShapeDtypeStruct((tokens, d), jnp.float32),
            jax.ShapeDtypeStruct((grid[0], 1, 1), jnp.float32),
        ],
        compiler_params=pltpu.CompilerParams(
            dimension_semantics=("parallel",),
        ),
    )(flat, codebooks)
    aux_loss = jnp.sum(aux_partials) / jnp.float32(tokens * d)
    return quant.reshape(b, n, d), aux_loss
